# baseline (speedup 1.0000x reference)
import functools

import jax
import jax.numpy as jnp
from jax import lax
from jax.experimental import pallas as pl
from jax.experimental.pallas import tpu as pltpu
from jax.experimental.pallas import tpu_sc as plsc

_N = 8192
_C = 512
_K = 20
_NEG = -3.0e38


_PREP_BLK = 512


def _prep_body(x_ref, w_ref, b_ref, q_ref, k_ref, a_ref, bb_ref, u_ref):
    p = jnp.dot(x_ref[...], w_ref[...], preferred_element_type=jnp.float32)
    p = p + b_ref[...]
    q_ref[...] = p[:, 0 * _C:1 * _C]
    k_ref[...] = p[:, 1 * _C:2 * _C]
    a_ref[...] = p[:, 2 * _C:3 * _C]
    bb_ref[...] = p[:, 3 * _C:4 * _C]
    u_ref[...] = p[:, 4 * _C:5 * _C]


def _prep(x, wcat, bcat):
    n_out = 5
    outs = pl.pallas_call(
        _prep_body,
        grid=(_N // _PREP_BLK,),
        in_specs=[
            pl.BlockSpec((_PREP_BLK, _C), lambda i: (i, 0)),
            pl.BlockSpec((_C, n_out * _C), lambda i: (0, 0)),
            pl.BlockSpec((1, n_out * _C), lambda i: (0, 0)),
        ],
        out_specs=[pl.BlockSpec((_PREP_BLK, _C), lambda i: (i, 0))] * n_out,
        out_shape=[jax.ShapeDtypeStruct((_N, _C), jnp.float32)] * n_out,
    )(x, wcat, bcat)
    return outs


_SIM_RB = 256
_CHUNK = 128
_NCHUNK = _N // _CHUNK


def _sim_body(k_ref, q_ref, sim_ref, thr_ref, cmax_ref):
    s = lax.dot_general(
        k_ref[...], q_ref[...], (((1,), (1,)), ((), ())),
        preferred_element_type=jnp.float32)
    sim_ref[...] = s
    cm = jnp.concatenate(
        [jnp.max(s[:, c * _CHUNK:(c + 1) * _CHUNK], axis=1, keepdims=True)
         for c in range(_NCHUNK)], axis=1)
    cmax_ref[...] = cm
    for _ in range(_K - 1):
        m = jnp.max(cm, axis=1, keepdims=True)
        cm = jnp.where(cm >= m, _NEG, cm)
    thr_ref[...] = jnp.max(cm, axis=1, keepdims=True)


def _sim(k, q):
    return pl.pallas_call(
        _sim_body,
        grid=(_N // _SIM_RB,),
        in_specs=[
            pl.BlockSpec((_SIM_RB, _C), lambda i: (i, 0)),
            pl.BlockSpec((_N, _C), lambda i: (0, 0)),
        ],
        out_specs=[
            pl.BlockSpec((_SIM_RB, _N), lambda i: (i, 0)),
            pl.BlockSpec((_SIM_RB, 1), lambda i: (i, 0)),
            pl.BlockSpec((_SIM_RB, _NCHUNK), lambda i: (i, 0)),
        ],
        out_shape=[
            jax.ShapeDtypeStruct((_N, _N), jnp.float32),
            jax.ShapeDtypeStruct((_N, 1), jnp.float32),
            jax.ShapeDtypeStruct((_N, _NCHUNK), jnp.float32),
        ],
    )(k, q)


_NC = 2
_NS = 16
_NW = _NC * _NS
_RPW = _N // _NW
_GC = 32


def _sc_body(sim_hbm, thr_hbm, cm_hbm, a_hbm, b_hbm, r_hbm,
             rowb0, rowb1, thrbuf, cmb0, cmb1, arow0, arow1, cvals, cidx,
             bbuf, acc0, acc1, sem, sem2,
             semr0, semr1, semc0, semc1, sema0, sema1, semw0, semw1):
    wid = lax.axis_index("s") * _NC + lax.axis_index("c")
    base = wid * _RPW
    pltpu.sync_copy(thr_hbm.at[pl.ds(base, _RPW)], thrbuf.at[pl.ds(0, _RPW)])
    iota16 = lax.iota(jnp.int32, 16)
    rowbufs = (rowb0, rowb1)
    cmbufs = (cmb0, cmb1)
    arows = (arow0, arow1)
    accbufs = (acc0, acc1)
    semr = (semr0, semr1)
    semc = (semc0, semc1)
    sema = (sema0, sema1)
    semw = (semw0, semw1)

    pltpu.async_copy(sim_hbm.at[base], rowbufs[0], semr[0])
    pltpu.async_copy(cm_hbm.at[base], cmbufs[0], semc[0])
    pltpu.async_copy(a_hbm.at[base], arows[0], sema[0])

    def pair_body(i, _carry):
      for par in (0, 1):
        row = base + 2 * i + par
        buf_sim = rowbufs[par]
        buf_cm = cmbufs[par]
        buf_a = arows[par]
        buf_acc = accbufs[par]
        pltpu.make_async_copy(sim_hbm.at[row], buf_sim, semr[par]).wait()
        pltpu.make_async_copy(cm_hbm.at[row], buf_cm, semc[par]).wait()
        pltpu.make_async_copy(a_hbm.at[row], buf_a, sema[par]).wait()

        def _pref():
            pltpu.async_copy(sim_hbm.at[row + 1], rowbufs[1 - par],
                             semr[1 - par])
            pltpu.async_copy(cm_hbm.at[row + 1], cmbufs[1 - par],
                             semc[1 - par])
            pltpu.async_copy(a_hbm.at[row + 1], arows[1 - par],
                             sema[1 - par])
        if par == 0:
            _pref()
        else:
            pl.when(i < _RPW // 2 - 1)(_pref)

        thr_s = thrbuf[pl.ds(2 * i + par, 16)][0]

        def cbody(g, cnt):
            cmv = buf_cm[pl.ds(g * 16, 16)]
            for l in range(16):
                cc = g * 16 + l

                def hit_fn(c, cc=cc):
                    for j in range(_CHUNK // 16):
                        v = buf_sim[pl.ds(cc * _CHUNK + j * 16, 16)]
                        msk = v >= thr_s
                        anyhit = jnp.any(msk)
                        cvals[pl.ds(c, 16)] = jnp.where(msk, v, _NEG)
                        cidx[pl.ds(c, 16)] = jnp.where(
                            msk, cc * _CHUNK + j * 16 + iota16, row)
                        c = c + jnp.where(anyhit, 16, 0)
                    return c

                cnt = lax.cond(cmv[l] >= thr_s, hit_fn, lambda c: c, cnt)
            return cnt

        cnt = lax.fori_loop(0, _NCHUNK // 16, cbody, jnp.int32(0))
        cvals[pl.ds(cnt, 16)] = jnp.full((16,), _NEG, jnp.float32)
        cidx[pl.ds(cnt, 16)] = jnp.zeros((16,), jnp.int32) + row
        nv = cnt // 16 + 1

        def sel_body(kk, carry):
            sel0, sel1, si0, si1 = carry

            def scan_body(j, bc):
                bv, bp = bc
                v = cvals[pl.ds(j * 16, 16)]
                p = j * 16 + iota16
                upd = v > bv
                return jnp.where(upd, v, bv), jnp.where(upd, p, bp)

            bv, bp = lax.fori_loop(
                0, nv, scan_body,
                (jnp.full((16,), -3.3e38, jnp.float32),
                 jnp.zeros((16,), jnp.int32)))
            m = jnp.max(bv)
            pos = jnp.max(jnp.where(bv == m, bp, -1))
            pbase = (pos // 16) * 16
            plane = pos - pbase
            vv = cvals[pl.ds(pbase, 16)]
            cvals[pl.ds(pbase, 16)] = jnp.where(iota16 == plane, _NEG, vv)
            ci = cidx[pl.ds(pos, 16)][0]
            in0 = kk < 16
            hit0 = jnp.logical_and(iota16 == kk, in0)
            hit1 = jnp.logical_and(iota16 == kk - 16, jnp.logical_not(in0))
            sel0 = jnp.where(hit0, m, sel0)
            si0 = jnp.where(hit0, ci, si0)
            sel1 = jnp.where(hit1, m, sel1)
            si1 = jnp.where(hit1, ci, si1)
            return sel0, sel1, si0, si1

        neg = jnp.full((16,), _NEG, jnp.float32)
        rsplat = jnp.zeros((16,), jnp.int32) + row
        sel0, sel1, si0, si1 = lax.fori_loop(
            0, _K, sel_body, (neg, neg, rsplat, rsplat))

        mm = jnp.maximum(jnp.max(sel0), jnp.max(sel1))
        e0 = jnp.exp(sel0 - mm)
        e1 = jnp.exp(sel1 - mm)
        ssum = jnp.sum(e0) + jnp.sum(e1)
        w0 = e0 / ssum
        w1 = e1 / ssum
        ws = [w0[l] for l in range(16)] + [w1[l] for l in range(_K - 16)]

        cp1 = pltpu.async_copy(b_hbm.at[si0], bbuf.at[pl.ds(0, 16)], sem)
        cp2 = pltpu.async_copy(b_hbm.at[si1], bbuf.at[pl.ds(16, 16)], sem2)
        cp1.wait()
        cp2.wait()

        def _drain():
            pltpu.make_async_copy(buf_acc, r_hbm.at[row], semw[par]).wait()
        pl.when(2 * i + par >= 2)(_drain)

        def ch_body(c2, _c):
            a = buf_a[pl.ds(c2 * 16, 16)]
            acc = jnp.zeros((16,), jnp.float32)
            for r2 in range(_K):
                b = bbuf[r2, pl.ds(c2 * 16, 16)]
                acc = acc + ws[r2] * jnp.maximum(a + b, 0.0)
            buf_acc[pl.ds(c2 * 16, 16)] = acc
            return _c

        lax.fori_loop(0, _C // 16, ch_body, 0, unroll=2)
        pltpu.async_copy(buf_acc, r_hbm.at[row], semw[par])
      return _carry

    lax.fori_loop(0, _RPW // 2, pair_body, 0)
    pltpu.make_async_copy(acc0, r_hbm.at[base + _RPW - 2], semw[0]).wait()
    pltpu.make_async_copy(acc1, r_hbm.at[base + _RPW - 1], semw[1]).wait()


def _sc_stage(sim, thr, cm, a, b):
    mesh = plsc.VectorSubcoreMesh(core_axis_name="c", subcore_axis_name="s")
    fn = functools.partial(
        pl.kernel,
        mesh=mesh,
        compiler_params=pltpu.CompilerParams(needs_layout_passes=False),
        out_type=jax.ShapeDtypeStruct((_N, _C), jnp.float32),
        scratch_types=[
            pltpu.VMEM((_N,), jnp.float32),
            pltpu.VMEM((_N,), jnp.float32),
            pltpu.VMEM((_RPW + 16,), jnp.float32),
            pltpu.VMEM((_NCHUNK,), jnp.float32),
            pltpu.VMEM((_NCHUNK,), jnp.float32),
            pltpu.VMEM((_C,), jnp.float32),
            pltpu.VMEM((_C,), jnp.float32),
            pltpu.VMEM((_N + 32,), jnp.float32),
            pltpu.VMEM((_N + 32,), jnp.int32),
            pltpu.VMEM((_GC, _C), jnp.float32),
            pltpu.VMEM((_C,), jnp.float32),
            pltpu.VMEM((_C,), jnp.float32),
        ] + [pltpu.SemaphoreType.DMA] * 10,
    )(_sc_body)
    return fn(sim, thr, cm, a, b)


_UPD_BLK = 512


def _upd_body(x_ref, r_ref, u_ref, w2_ref, b2_ref, uw1b_ref, uw2_ref,
              ub2_ref, o_ref):
    h = jnp.dot(r_ref[...], w2_ref[...],
                preferred_element_type=jnp.float32) + b2_ref[...]
    t = jnp.maximum(
        u_ref[...] + jnp.dot(h, uw1b_ref[...],
                             preferred_element_type=jnp.float32), 0.0)
    o_ref[...] = x_ref[...] + jnp.dot(
        t, uw2_ref[...], preferred_element_type=jnp.float32) + ub2_ref[...]


def _upd(x, r, u, w2, b2, uw1b, uw2, ub2):
    row_spec = pl.BlockSpec((_UPD_BLK, _C), lambda i: (i, 0))
    w_spec = pl.BlockSpec((_C, _C), lambda i: (0, 0))
    b_spec = pl.BlockSpec((1, _C), lambda i: (0, 0))
    return pl.pallas_call(
        _upd_body,
        grid=(_N // _UPD_BLK,),
        in_specs=[row_spec, row_spec, row_spec, w_spec, b_spec, w_spec,
                  w_spec, b_spec],
        out_specs=row_spec,
        out_shape=jax.ShapeDtypeStruct((_N, _C), jnp.float32),
    )(x, r, u, w2, b2, uw1b, uw2, ub2)


def kernel(x, Wq, bq, Wk, bk, msg_W1, msg_b1, msg_W2, msg_b2,
           upd_W1, upd_b1, upd_W2, upd_b2):
    wcat = jnp.concatenate(
        [Wq, Wk, msg_W1[:_C], msg_W1[_C:], upd_W1[:_C]], axis=1)
    bcat = jnp.concatenate(
        [bq, bk, msg_b1, jnp.zeros_like(msg_b1), upd_b1]).reshape(1, 5 * _C)
    q, k, a, b, u = _prep(x, wcat, bcat)
    sim, thr, cm = _sim(k, q)
    r = _sc_stage(sim, thr.reshape(_N), cm, a, b)
    return _upd(x, r, u, msg_W2, msg_b2.reshape(1, _C), upd_W1[_C:],
                upd_W2, upd_b2.reshape(1, _C))


# speedup vs baseline: 1.1981x; 1.1981x over previous
import functools

import jax
import jax.numpy as jnp
from jax import lax
from jax.experimental import pallas as pl
from jax.experimental.pallas import tpu as pltpu
from jax.experimental.pallas import tpu_sc as plsc

_N = 8192
_C = 512
_K = 20
_NEG = -3.0e38


_PREP_BLK = 512


def _prep_body(x_ref, w_ref, b_ref, q_ref, k_ref, a_ref, bb_ref, u_ref):
    p = jnp.dot(x_ref[...], w_ref[...], preferred_element_type=jnp.float32)
    p = p + b_ref[...]
    q_ref[...] = p[:, 0 * _C:1 * _C]
    k_ref[...] = p[:, 1 * _C:2 * _C]
    a_ref[...] = p[:, 2 * _C:3 * _C]
    bb_ref[...] = p[:, 3 * _C:4 * _C]
    u_ref[...] = p[:, 4 * _C:5 * _C]


def _prep(x, wcat, bcat):
    n_out = 5
    outs = pl.pallas_call(
        _prep_body,
        grid=(_N // _PREP_BLK,),
        in_specs=[
            pl.BlockSpec((_PREP_BLK, _C), lambda i: (i, 0)),
            pl.BlockSpec((_C, n_out * _C), lambda i: (0, 0)),
            pl.BlockSpec((1, n_out * _C), lambda i: (0, 0)),
        ],
        out_specs=[pl.BlockSpec((_PREP_BLK, _C), lambda i: (i, 0))] * n_out,
        out_shape=[jax.ShapeDtypeStruct((_N, _C), jnp.float32)] * n_out,
    )(x, wcat, bcat)
    return outs


_SIM_RB = 256
_CHUNK = 128
_NCHUNK = _N // _CHUNK


def _sim_body(k_ref, q_ref, sim_ref, thr_ref, cmax_ref):
    s = lax.dot_general(
        k_ref[...], q_ref[...], (((1,), (1,)), ((), ())),
        preferred_element_type=jnp.float32)
    sim_ref[...] = s
    cm = jnp.concatenate(
        [jnp.max(s[:, c * _CHUNK:(c + 1) * _CHUNK], axis=1, keepdims=True)
         for c in range(_NCHUNK)], axis=1)
    cmax_ref[...] = cm
    for _ in range(_K - 1):
        m = jnp.max(cm, axis=1, keepdims=True)
        cm = jnp.where(cm >= m, _NEG, cm)
    thr_ref[...] = jnp.max(cm, axis=1, keepdims=True)


def _sim(k, q):
    return pl.pallas_call(
        _sim_body,
        grid=(_N // _SIM_RB,),
        in_specs=[
            pl.BlockSpec((_SIM_RB, _C), lambda i: (i, 0)),
            pl.BlockSpec((_N, _C), lambda i: (0, 0)),
        ],
        out_specs=[
            pl.BlockSpec((_SIM_RB, _N), lambda i: (i, 0)),
            pl.BlockSpec((_SIM_RB, 1), lambda i: (i, 0)),
            pl.BlockSpec((_SIM_RB, _NCHUNK), lambda i: (i, 0)),
        ],
        out_shape=[
            jax.ShapeDtypeStruct((_N, _N), jnp.float32),
            jax.ShapeDtypeStruct((_N, 1), jnp.float32),
            jax.ShapeDtypeStruct((_N, _NCHUNK), jnp.float32),
        ],
    )(k, q)


_NC = 2
_NS = 16
_NW = _NC * _NS
_RPW = _N // _NW
_GC = 32


def _sc_body(sim_hbm, thr_hbm, cm_hbm, a_hbm, b_hbm, r_hbm,
             rowb0, rowb1, thrbuf, cmb0, cmb1, arow0, arow1, cvals, cidx,
             bbuf, acc0, acc1, sem, sem2,
             semr0, semr1, semc0, semc1, sema0, sema1, semw0, semw1):
    wid = lax.axis_index("s") * _NC + lax.axis_index("c")
    base = wid * _RPW
    pltpu.sync_copy(thr_hbm.at[pl.ds(base, _RPW)], thrbuf.at[pl.ds(0, _RPW)])
    iota16 = lax.iota(jnp.int32, 16)
    rowbufs = (rowb0, rowb1)
    cmbufs = (cmb0, cmb1)
    arows = (arow0, arow1)
    accbufs = (acc0, acc1)
    semr = (semr0, semr1)
    semc = (semc0, semc1)
    sema = (sema0, sema1)
    semw = (semw0, semw1)

    pltpu.async_copy(sim_hbm.at[base], rowbufs[0], semr[0])
    pltpu.async_copy(cm_hbm.at[base], cmbufs[0], semc[0])
    pltpu.async_copy(a_hbm.at[base], arows[0], sema[0])

    def pair_body(i, _carry):
      for par in (0, 1):
        row = base + 2 * i + par
        buf_sim = rowbufs[par]
        buf_cm = cmbufs[par]
        buf_a = arows[par]
        buf_acc = accbufs[par]
        pltpu.make_async_copy(sim_hbm.at[row], buf_sim, semr[par]).wait()
        pltpu.make_async_copy(cm_hbm.at[row], buf_cm, semc[par]).wait()
        pltpu.make_async_copy(a_hbm.at[row], buf_a, sema[par]).wait()

        def _pref():
            pltpu.async_copy(sim_hbm.at[row + 1], rowbufs[1 - par],
                             semr[1 - par])
            pltpu.async_copy(cm_hbm.at[row + 1], cmbufs[1 - par],
                             semc[1 - par])
            pltpu.async_copy(a_hbm.at[row + 1], arows[1 - par],
                             sema[1 - par])
        if par == 0:
            _pref()
        else:
            pl.when(i < _RPW // 2 - 1)(_pref)

        thr_s = thrbuf[pl.ds(2 * i + par, 16)][0]

        def cbody(cc, cnt):
            v = buf_sim[pl.ds(cc * 16, 16)]
            msk = v >= thr_s
            anyhit = jnp.any(msk)
            cvals[pl.ds(cnt, 16)] = jnp.where(msk, v, _NEG)
            cidx[pl.ds(cnt, 16)] = jnp.where(msk, cc * 16 + iota16, row)
            return cnt + jnp.where(anyhit, 16, 0)

        cnt = lax.fori_loop(0, _N // 16, cbody, jnp.int32(0), unroll=8)
        cvals[pl.ds(cnt, 16)] = jnp.full((16,), _NEG, jnp.float32)
        cidx[pl.ds(cnt, 16)] = jnp.zeros((16,), jnp.int32) + row
        nv = cnt // 16 + 1

        def sel_body(kk, carry):
            sel0, sel1, si0, si1 = carry

            def scan_body(j, bc):
                bv, bp = bc
                v = cvals[pl.ds(j * 16, 16)]
                p = j * 16 + iota16
                upd = v > bv
                return jnp.where(upd, v, bv), jnp.where(upd, p, bp)

            bv, bp = lax.fori_loop(
                0, nv, scan_body,
                (jnp.full((16,), -3.3e38, jnp.float32),
                 jnp.zeros((16,), jnp.int32)))
            m = jnp.max(bv)
            pos = jnp.max(jnp.where(bv == m, bp, -1))
            pbase = (pos // 16) * 16
            plane = pos - pbase
            vv = cvals[pl.ds(pbase, 16)]
            cvals[pl.ds(pbase, 16)] = jnp.where(iota16 == plane, _NEG, vv)
            ci = cidx[pl.ds(pos, 16)][0]
            in0 = kk < 16
            hit0 = jnp.logical_and(iota16 == kk, in0)
            hit1 = jnp.logical_and(iota16 == kk - 16, jnp.logical_not(in0))
            sel0 = jnp.where(hit0, m, sel0)
            si0 = jnp.where(hit0, ci, si0)
            sel1 = jnp.where(hit1, m, sel1)
            si1 = jnp.where(hit1, ci, si1)
            return sel0, sel1, si0, si1

        neg = jnp.full((16,), _NEG, jnp.float32)
        rsplat = jnp.zeros((16,), jnp.int32) + row
        sel0, sel1, si0, si1 = lax.fori_loop(
            0, _K, sel_body, (neg, neg, rsplat, rsplat))

        mm = jnp.maximum(jnp.max(sel0), jnp.max(sel1))
        e0 = jnp.exp(sel0 - mm)
        e1 = jnp.exp(sel1 - mm)
        ssum = jnp.sum(e0) + jnp.sum(e1)
        w0 = e0 / ssum
        w1 = e1 / ssum
        ws = [w0[l] for l in range(16)] + [w1[l] for l in range(_K - 16)]

        cp1 = pltpu.async_copy(b_hbm.at[si0], bbuf.at[pl.ds(0, 16)], sem)
        cp2 = pltpu.async_copy(b_hbm.at[si1], bbuf.at[pl.ds(16, 16)], sem2)
        cp1.wait()
        cp2.wait()

        def _drain():
            pltpu.make_async_copy(buf_acc, r_hbm.at[row], semw[par]).wait()
        pl.when(2 * i + par >= 2)(_drain)

        def ch_body(c2, _c):
            a = buf_a[pl.ds(c2 * 16, 16)]
            acc = jnp.zeros((16,), jnp.float32)
            for r2 in range(_K):
                b = bbuf[r2, pl.ds(c2 * 16, 16)]
                acc = acc + ws[r2] * jnp.maximum(a + b, 0.0)
            buf_acc[pl.ds(c2 * 16, 16)] = acc
            return _c

        lax.fori_loop(0, _C // 16, ch_body, 0, unroll=2)
        pltpu.async_copy(buf_acc, r_hbm.at[row], semw[par])
      return _carry

    lax.fori_loop(0, _RPW // 2, pair_body, 0)
    pltpu.make_async_copy(acc0, r_hbm.at[base + _RPW - 2], semw[0]).wait()
    pltpu.make_async_copy(acc1, r_hbm.at[base + _RPW - 1], semw[1]).wait()


def _sc_stage(sim, thr, cm, a, b):
    mesh = plsc.VectorSubcoreMesh(core_axis_name="c", subcore_axis_name="s")
    fn = functools.partial(
        pl.kernel,
        mesh=mesh,
        compiler_params=pltpu.CompilerParams(needs_layout_passes=False),
        out_type=jax.ShapeDtypeStruct((_N, _C), jnp.float32),
        scratch_types=[
            pltpu.VMEM((_N,), jnp.float32),
            pltpu.VMEM((_N,), jnp.float32),
            pltpu.VMEM((_RPW + 16,), jnp.float32),
            pltpu.VMEM((_NCHUNK,), jnp.float32),
            pltpu.VMEM((_NCHUNK,), jnp.float32),
            pltpu.VMEM((_C,), jnp.float32),
            pltpu.VMEM((_C,), jnp.float32),
            pltpu.VMEM((_N + 32,), jnp.float32),
            pltpu.VMEM((_N + 32,), jnp.int32),
            pltpu.VMEM((_GC, _C), jnp.float32),
            pltpu.VMEM((_C,), jnp.float32),
            pltpu.VMEM((_C,), jnp.float32),
        ] + [pltpu.SemaphoreType.DMA] * 10,
    )(_sc_body)
    return fn(sim, thr, cm, a, b)


_UPD_BLK = 512


def _upd_body(x_ref, r_ref, u_ref, w2_ref, b2_ref, uw1b_ref, uw2_ref,
              ub2_ref, o_ref):
    h = jnp.dot(r_ref[...], w2_ref[...],
                preferred_element_type=jnp.float32) + b2_ref[...]
    t = jnp.maximum(
        u_ref[...] + jnp.dot(h, uw1b_ref[...],
                             preferred_element_type=jnp.float32), 0.0)
    o_ref[...] = x_ref[...] + jnp.dot(
        t, uw2_ref[...], preferred_element_type=jnp.float32) + ub2_ref[...]


def _upd(x, r, u, w2, b2, uw1b, uw2, ub2):
    row_spec = pl.BlockSpec((_UPD_BLK, _C), lambda i: (i, 0))
    w_spec = pl.BlockSpec((_C, _C), lambda i: (0, 0))
    b_spec = pl.BlockSpec((1, _C), lambda i: (0, 0))
    return pl.pallas_call(
        _upd_body,
        grid=(_N // _UPD_BLK,),
        in_specs=[row_spec, row_spec, row_spec, w_spec, b_spec, w_spec,
                  w_spec, b_spec],
        out_specs=row_spec,
        out_shape=jax.ShapeDtypeStruct((_N, _C), jnp.float32),
    )(x, r, u, w2, b2, uw1b, uw2, ub2)


def kernel(x, Wq, bq, Wk, bk, msg_W1, msg_b1, msg_W2, msg_b2,
           upd_W1, upd_b1, upd_W2, upd_b2):
    wcat = jnp.concatenate(
        [Wq, Wk, msg_W1[:_C], msg_W1[_C:], upd_W1[:_C]], axis=1)
    bcat = jnp.concatenate(
        [bq, bk, msg_b1, jnp.zeros_like(msg_b1), upd_b1]).reshape(1, 5 * _C)
    q, k, a, b, u = _prep(x, wcat, bcat)
    sim, thr, cm = _sim(k, q)
    r = _sc_stage(sim, thr.reshape(_N), cm, a, b)
    return _upd(x, r, u, msg_W2, msg_b2.reshape(1, _C), upd_W1[_C:],
                upd_W2, upd_b2.reshape(1, _C))


# speedup vs baseline: 1.2989x; 1.0842x over previous
import functools

import jax
import jax.numpy as jnp
from jax import lax
from jax.experimental import pallas as pl
from jax.experimental.pallas import tpu as pltpu
from jax.experimental.pallas import tpu_sc as plsc

_N = 8192
_C = 512
_K = 20
_NEG = -3.0e38


_PREP_BLK = 512


def _prep_body(x_ref, w_ref, b_ref, q_ref, k_ref, a_ref, bb_ref, u_ref):
    p = jnp.dot(x_ref[...], w_ref[...], preferred_element_type=jnp.float32)
    p = p + b_ref[...]
    q_ref[...] = p[:, 0 * _C:1 * _C]
    k_ref[...] = p[:, 1 * _C:2 * _C]
    a_ref[...] = p[:, 2 * _C:3 * _C]
    bb_ref[...] = p[:, 3 * _C:4 * _C]
    u_ref[...] = p[:, 4 * _C:5 * _C]


def _prep(x, wcat, bcat):
    n_out = 5
    outs = pl.pallas_call(
        _prep_body,
        grid=(_N // _PREP_BLK,),
        in_specs=[
            pl.BlockSpec((_PREP_BLK, _C), lambda i: (i, 0)),
            pl.BlockSpec((_C, n_out * _C), lambda i: (0, 0)),
            pl.BlockSpec((1, n_out * _C), lambda i: (0, 0)),
        ],
        out_specs=[pl.BlockSpec((_PREP_BLK, _C), lambda i: (i, 0))] * n_out,
        out_shape=[jax.ShapeDtypeStruct((_N, _C), jnp.float32)] * n_out,
    )(x, wcat, bcat)
    return outs


_SIM_RB = 256
_CHUNK = 128
_NCHUNK = _N // _CHUNK


def _sim_body(k_ref, q_ref, sim_ref, thr_ref, cmax_ref):
    s = lax.dot_general(
        k_ref[...], q_ref[...], (((1,), (1,)), ((), ())),
        preferred_element_type=jnp.float32)
    sim_ref[...] = s
    cm = jnp.concatenate(
        [jnp.max(s[:, c * _CHUNK:(c + 1) * _CHUNK], axis=1, keepdims=True)
         for c in range(_NCHUNK)], axis=1)
    cmax_ref[...] = cm
    for _ in range(_K - 1):
        m = jnp.max(cm, axis=1, keepdims=True)
        cm = jnp.where(cm >= m, _NEG, cm)
    thr_ref[...] = jnp.max(cm, axis=1, keepdims=True)


def _sim(k, q):
    return pl.pallas_call(
        _sim_body,
        grid=(_N // _SIM_RB,),
        in_specs=[
            pl.BlockSpec((_SIM_RB, _C), lambda i: (i, 0)),
            pl.BlockSpec((_N, _C), lambda i: (0, 0)),
        ],
        out_specs=[
            pl.BlockSpec((_SIM_RB, _N), lambda i: (i, 0)),
            pl.BlockSpec((_SIM_RB, 1), lambda i: (i, 0)),
            pl.BlockSpec((_SIM_RB, _NCHUNK), lambda i: (i, 0)),
        ],
        out_shape=[
            jax.ShapeDtypeStruct((_N, _N), jnp.float32),
            jax.ShapeDtypeStruct((_N, 1), jnp.float32),
            jax.ShapeDtypeStruct((_N, _NCHUNK), jnp.float32),
        ],
    )(k, q)


_NC = 2
_NS = 16
_NW = _NC * _NS
_RPW = _N // _NW
_GC = 32


def _sc_body(sim_hbm, thr_hbm, cm_hbm, a_hbm, b_hbm, r_hbm,
             rowb0, rowb1, thrbuf, cmb0, cmb1, arow0, arow1, cbuf,
             bbuf, acc0, acc1, sem, sem2,
             semr0, semr1, semc0, semc1, sema0, sema1, semw0, semw1):
    wid = lax.axis_index("s") * _NC + lax.axis_index("c")
    base = wid * _RPW
    pltpu.sync_copy(thr_hbm.at[pl.ds(base, _RPW)], thrbuf.at[pl.ds(0, _RPW)])
    iota16 = lax.iota(jnp.int32, 16)
    rowbufs = (rowb0, rowb1)
    cmbufs = (cmb0, cmb1)
    arows = (arow0, arow1)
    accbufs = (acc0, acc1)
    semr = (semr0, semr1)
    semc = (semc0, semc1)
    sema = (sema0, sema1)
    semw = (semw0, semw1)

    pltpu.async_copy(sim_hbm.at[base], rowbufs[0], semr[0])
    pltpu.async_copy(cm_hbm.at[base], cmbufs[0], semc[0])
    pltpu.async_copy(a_hbm.at[base], arows[0], sema[0])

    def pair_body(i, _carry):
      for par in (0, 1):
        row = base + 2 * i + par
        buf_sim = rowbufs[par]
        buf_cm = cmbufs[par]
        buf_a = arows[par]
        buf_acc = accbufs[par]
        pltpu.make_async_copy(sim_hbm.at[row], buf_sim, semr[par]).wait()
        pltpu.make_async_copy(cm_hbm.at[row], buf_cm, semc[par]).wait()
        pltpu.make_async_copy(a_hbm.at[row], buf_a, sema[par]).wait()

        def _pref():
            pltpu.async_copy(sim_hbm.at[row + 1], rowbufs[1 - par],
                             semr[1 - par])
            pltpu.async_copy(cm_hbm.at[row + 1], cmbufs[1 - par],
                             semc[1 - par])
            pltpu.async_copy(a_hbm.at[row + 1], arows[1 - par],
                             sema[1 - par])
        if par == 0:
            _pref()
        else:
            pl.when(i < _RPW // 2 - 1)(_pref)

        thr_s = thrbuf[pl.ds(2 * i + par, 16)][0]

        dump = jnp.int32(_N + 40)

        def cbody(cc, cnt_vec):
            v = buf_sim[pl.ds(cc * 16, 16)]
            msk = v >= thr_s
            mono = plsc.bitcast(v, jnp.uint32) ^ jnp.where(
                v < 0.0, jnp.uint32(0xFFFFFFFF), jnp.uint32(0x80000000))
            col = cc * 16 + iota16
            comb = (mono & jnp.uint32(0xFFFFE000)) | col.astype(jnp.uint32)
            ones = jnp.where(msk, jnp.int32(1), jnp.int32(0))
            cs = plsc.cumsum(ones)
            pos = jnp.where(msk, cnt_vec + cs - 1, dump)
            plsc.store_scatter(cbuf, [pos], plsc.bitcast(comb, jnp.int32))
            return cnt_vec + plsc.all_reduce_population_count(msk)

        cnt_vec = lax.fori_loop(0, _N // 16, cbody,
                                jnp.zeros((16,), jnp.int32), unroll=8)
        cnt = cnt_vec[0]
        cbuf[pl.ds(cnt, 16)] = jnp.zeros((16,), jnp.int32)
        nv = cnt // 16 + 1

        rsplat = jnp.zeros((16,), jnp.int32) + row

        def sel_body(kk, carry):
            last, si0, si1 = carry

            def scan_body(j, bc):
                vu = plsc.bitcast(cbuf[pl.ds(j * 16, 16)], jnp.uint32)
                vu = jnp.where(vu == last, jnp.uint32(0), vu)
                cbuf[pl.ds(j * 16, 16)] = plsc.bitcast(vu, jnp.int32)
                return jnp.maximum(bc, vu)

            bc = lax.fori_loop(0, nv, scan_body, jnp.zeros((16,), jnp.uint32))
            best = jnp.max(bc)
            col = (best & jnp.uint32(0x1FFF)).astype(jnp.int32)
            hit0 = jnp.logical_and(iota16 == kk, kk < 16)
            hit1 = jnp.logical_and(iota16 == kk - 16, kk >= 16)
            si0 = jnp.where(hit0, col, si0)
            si1 = jnp.where(hit1, col, si1)
            return best, si0, si1

        _, si0, si1 = lax.fori_loop(
            0, _K, sel_body, (jnp.uint32(0), rsplat, rsplat))

        sel0 = plsc.load_gather(buf_sim, [si0])
        sel1 = jnp.where(iota16 < _K - 16,
                         plsc.load_gather(buf_sim, [si1]), _NEG)

        mm = jnp.maximum(jnp.max(sel0), jnp.max(sel1))
        e0 = jnp.exp(sel0 - mm)
        e1 = jnp.exp(sel1 - mm)
        ssum = jnp.sum(e0) + jnp.sum(e1)
        w0 = e0 / ssum
        w1 = e1 / ssum
        ws = [w0[l] for l in range(16)] + [w1[l] for l in range(_K - 16)]

        cp1 = pltpu.async_copy(b_hbm.at[si0], bbuf.at[pl.ds(0, 16)], sem)
        cp2 = pltpu.async_copy(b_hbm.at[si1], bbuf.at[pl.ds(16, 16)], sem2)
        cp1.wait()
        cp2.wait()

        def _drain():
            pltpu.make_async_copy(buf_acc, r_hbm.at[row], semw[par]).wait()
        pl.when(2 * i + par >= 2)(_drain)

        def ch_body(c2, _c):
            a = buf_a[pl.ds(c2 * 16, 16)]
            acc = jnp.zeros((16,), jnp.float32)
            for r2 in range(_K):
                b = bbuf[r2, pl.ds(c2 * 16, 16)]
                acc = acc + ws[r2] * jnp.maximum(a + b, 0.0)
            buf_acc[pl.ds(c2 * 16, 16)] = acc
            return _c

        lax.fori_loop(0, _C // 16, ch_body, 0, unroll=2)
        pltpu.async_copy(buf_acc, r_hbm.at[row], semw[par])
      return _carry

    lax.fori_loop(0, _RPW // 2, pair_body, 0)
    pltpu.make_async_copy(acc0, r_hbm.at[base + _RPW - 2], semw[0]).wait()
    pltpu.make_async_copy(acc1, r_hbm.at[base + _RPW - 1], semw[1]).wait()


def _sc_stage(sim, thr, cm, a, b):
    mesh = plsc.VectorSubcoreMesh(core_axis_name="c", subcore_axis_name="s")
    fn = functools.partial(
        pl.kernel,
        mesh=mesh,
        compiler_params=pltpu.CompilerParams(needs_layout_passes=False),
        out_type=jax.ShapeDtypeStruct((_N, _C), jnp.float32),
        scratch_types=[
            pltpu.VMEM((_N,), jnp.float32),
            pltpu.VMEM((_N,), jnp.float32),
            pltpu.VMEM((_RPW + 16,), jnp.float32),
            pltpu.VMEM((_NCHUNK,), jnp.float32),
            pltpu.VMEM((_NCHUNK,), jnp.float32),
            pltpu.VMEM((_C,), jnp.float32),
            pltpu.VMEM((_C,), jnp.float32),
            pltpu.VMEM((_N + 64,), jnp.int32),
            pltpu.VMEM((_GC, _C), jnp.float32),
            pltpu.VMEM((_C,), jnp.float32),
            pltpu.VMEM((_C,), jnp.float32),
        ] + [pltpu.SemaphoreType.DMA] * 10,
    )(_sc_body)
    return fn(sim, thr, cm, a, b)


_UPD_BLK = 512


def _upd_body(x_ref, r_ref, u_ref, w2_ref, b2_ref, uw1b_ref, uw2_ref,
              ub2_ref, o_ref):
    h = jnp.dot(r_ref[...], w2_ref[...],
                preferred_element_type=jnp.float32) + b2_ref[...]
    t = jnp.maximum(
        u_ref[...] + jnp.dot(h, uw1b_ref[...],
                             preferred_element_type=jnp.float32), 0.0)
    o_ref[...] = x_ref[...] + jnp.dot(
        t, uw2_ref[...], preferred_element_type=jnp.float32) + ub2_ref[...]


def _upd(x, r, u, w2, b2, uw1b, uw2, ub2):
    row_spec = pl.BlockSpec((_UPD_BLK, _C), lambda i: (i, 0))
    w_spec = pl.BlockSpec((_C, _C), lambda i: (0, 0))
    b_spec = pl.BlockSpec((1, _C), lambda i: (0, 0))
    return pl.pallas_call(
        _upd_body,
        grid=(_N // _UPD_BLK,),
        in_specs=[row_spec, row_spec, row_spec, w_spec, b_spec, w_spec,
                  w_spec, b_spec],
        out_specs=row_spec,
        out_shape=jax.ShapeDtypeStruct((_N, _C), jnp.float32),
    )(x, r, u, w2, b2, uw1b, uw2, ub2)


def kernel(x, Wq, bq, Wk, bk, msg_W1, msg_b1, msg_W2, msg_b2,
           upd_W1, upd_b1, upd_W2, upd_b2):
    wcat = jnp.concatenate(
        [Wq, Wk, msg_W1[:_C], msg_W1[_C:], upd_W1[:_C]], axis=1)
    bcat = jnp.concatenate(
        [bq, bk, msg_b1, jnp.zeros_like(msg_b1), upd_b1]).reshape(1, 5 * _C)
    q, k, a, b, u = _prep(x, wcat, bcat)
    sim, thr, cm = _sim(k, q)
    r = _sc_stage(sim, thr.reshape(_N), cm, a, b)
    return _upd(x, r, u, msg_W2, msg_b2.reshape(1, _C), upd_W1[_C:],
                upd_W2, upd_b2.reshape(1, _C))


# speedup vs baseline: 1.5354x; 1.1821x over previous
import functools

import jax
import jax.numpy as jnp
from jax import lax
from jax.experimental import pallas as pl
from jax.experimental.pallas import tpu as pltpu
from jax.experimental.pallas import tpu_sc as plsc

_N = 8192
_C = 512
_K = 20
_NEG = -3.0e38


_PREP_BLK = 512


def _prep_body(x_ref, w_ref, b_ref, q_ref, k_ref, a_ref, bb_ref, u_ref):
    p = jnp.dot(x_ref[...], w_ref[...], preferred_element_type=jnp.float32)
    p = p + b_ref[...]
    q_ref[...] = p[:, 0 * _C:1 * _C]
    k_ref[...] = p[:, 1 * _C:2 * _C]
    a_ref[...] = p[:, 2 * _C:3 * _C]
    bb_ref[...] = p[:, 3 * _C:4 * _C]
    u_ref[...] = p[:, 4 * _C:5 * _C]


def _prep(x, wcat, bcat):
    n_out = 5
    outs = pl.pallas_call(
        _prep_body,
        grid=(_N // _PREP_BLK,),
        in_specs=[
            pl.BlockSpec((_PREP_BLK, _C), lambda i: (i, 0)),
            pl.BlockSpec((_C, n_out * _C), lambda i: (0, 0)),
            pl.BlockSpec((1, n_out * _C), lambda i: (0, 0)),
        ],
        out_specs=[pl.BlockSpec((_PREP_BLK, _C), lambda i: (i, 0))] * n_out,
        out_shape=[jax.ShapeDtypeStruct((_N, _C), jnp.float32)] * n_out,
    )(x, wcat, bcat)
    return outs


_SIM_RB = 256
_CHUNK = 128
_NCHUNK = _N // _CHUNK


def _sim_body(k_ref, q_ref, sim_ref, thr_ref, cmax_ref):
    s = lax.dot_general(
        k_ref[...], q_ref[...], (((1,), (1,)), ((), ())),
        preferred_element_type=jnp.float32)
    sim_ref[...] = s
    cm = jnp.concatenate(
        [jnp.max(s[:, c * _CHUNK:(c + 1) * _CHUNK], axis=1, keepdims=True)
         for c in range(_NCHUNK)], axis=1)
    cmax_ref[...] = cm
    for _ in range(_K - 1):
        m = jnp.max(cm, axis=1, keepdims=True)
        cm = jnp.where(cm >= m, _NEG, cm)
    thr_ref[...] = jnp.max(cm, axis=1, keepdims=True)


def _sim(k, q):
    return pl.pallas_call(
        _sim_body,
        grid=(_N // _SIM_RB,),
        in_specs=[
            pl.BlockSpec((_SIM_RB, _C), lambda i: (i, 0)),
            pl.BlockSpec((_N, _C), lambda i: (0, 0)),
        ],
        out_specs=[
            pl.BlockSpec((_SIM_RB, _N), lambda i: (i, 0)),
            pl.BlockSpec((_SIM_RB, 1), lambda i: (i, 0)),
            pl.BlockSpec((_SIM_RB, _NCHUNK), lambda i: (i, 0)),
        ],
        out_shape=[
            jax.ShapeDtypeStruct((_N, _N), jnp.float32),
            jax.ShapeDtypeStruct((_N, 1), jnp.float32),
            jax.ShapeDtypeStruct((_N, _NCHUNK), jnp.float32),
        ],
    )(k, q)


_NC = 2
_NS = 16
_NW = _NC * _NS
_RPW = _N // _NW
_GC = 32


def _sc_body(sim_hbm, thr_hbm, cm_hbm, a_hbm, b_hbm, r_hbm,
             rowb0, rowb1, thrbuf, cmb0, cmb1, arow0, arow1, cbuf,
             bbuf0, bbuf1, accbuf, semb00, semb01, semb10, semb11,
             semr0, semr1, semc0, semc1, sema0, sema1):
    wid = lax.axis_index("s") * _NC + lax.axis_index("c")
    base = wid * _RPW
    pltpu.sync_copy(thr_hbm.at[pl.ds(base, _RPW)], thrbuf.at[pl.ds(0, _RPW)])
    iota16 = lax.iota(jnp.int32, 16)
    rowbufs = (rowb0, rowb1)
    cmbufs = (cmb0, cmb1)
    arows = (arow0, arow1)
    bbufs = (bbuf0, bbuf1)
    semr = (semr0, semr1)
    semc = (semc0, semc1)
    sema = (sema0, sema1)
    semb = ((semb00, semb01), (semb10, semb11))
    dummy_idx = jnp.zeros((16,), jnp.int32)

    def agg_write(gpar, rowm1, ws):
        pltpu.make_async_copy(b_hbm.at[dummy_idx],
                              bbufs[gpar].at[pl.ds(0, 16)],
                              semb[gpar][0]).wait()
        pltpu.make_async_copy(b_hbm.at[dummy_idx],
                              bbufs[gpar].at[pl.ds(16, 16)],
                              semb[gpar][1]).wait()
        pltpu.make_async_copy(a_hbm.at[rowm1], arows[gpar],
                              sema[gpar]).wait()

        def ch_body(c2, _c):
            a = arows[gpar][pl.ds(c2 * 16, 16)]
            acc = jnp.zeros((16,), jnp.float32)
            for r2 in range(_K):
                b = bbufs[gpar][r2, pl.ds(c2 * 16, 16)]
                acc = acc + ws[r2] * jnp.maximum(a + b, 0.0)
            accbuf[pl.ds(c2 * 16, 16)] = acc
            return _c

        lax.fori_loop(0, _C // 16, ch_body, 0, unroll=2)
        pltpu.sync_copy(accbuf, r_hbm.at[rowm1])

    pltpu.async_copy(sim_hbm.at[base], rowbufs[0], semr[0])
    pltpu.async_copy(cm_hbm.at[base], cmbufs[0], semc[0])

    def pair_body(i, carry):
      ws = carry
      for par in (0, 1):
        row = base + 2 * i + par
        buf_sim = rowbufs[par]
        buf_cm = cmbufs[par]
        pltpu.make_async_copy(sim_hbm.at[row], buf_sim, semr[par]).wait()
        pltpu.make_async_copy(cm_hbm.at[row], buf_cm, semc[par]).wait()

        def _pref():
            pltpu.async_copy(sim_hbm.at[row + 1], rowbufs[1 - par],
                             semr[1 - par])
            pltpu.async_copy(cm_hbm.at[row + 1], cmbufs[1 - par],
                             semc[1 - par])
        if par == 0:
            _pref()
        else:
            pl.when(i < _RPW // 2 - 1)(_pref)

        thr_s = thrbuf[pl.ds(2 * i + par, 16)][0]

        dump = jnp.int32(_N + 40)

        def cbody(cc, cnt_vec):
            v = buf_sim[pl.ds(cc * 16, 16)]
            msk = v >= thr_s
            mono = plsc.bitcast(v, jnp.uint32) ^ jnp.where(
                v < 0.0, jnp.uint32(0xFFFFFFFF), jnp.uint32(0x80000000))
            col = cc * 16 + iota16
            comb = (mono & jnp.uint32(0xFFFFE000)) | col.astype(jnp.uint32)
            ones = jnp.where(msk, jnp.int32(1), jnp.int32(0))
            cs = plsc.cumsum(ones)
            pos = jnp.where(msk, cnt_vec + cs - 1, dump)
            plsc.store_scatter(cbuf, [pos], plsc.bitcast(comb, jnp.int32))
            return cnt_vec + plsc.all_reduce_population_count(msk)

        cnt_vec = lax.fori_loop(0, _N // 16, cbody,
                                jnp.zeros((16,), jnp.int32), unroll=8)
        cnt = cnt_vec[0]
        cbuf[pl.ds(cnt, 16)] = jnp.zeros((16,), jnp.int32)
        nv = cnt // 16 + 1

        rsplat = jnp.zeros((16,), jnp.int32) + row

        def sel_body(kk, carry):
            last, si0, si1 = carry

            def scan_body(j, bc):
                vu = plsc.bitcast(cbuf[pl.ds(j * 16, 16)], jnp.uint32)
                vu = jnp.where(vu == last, jnp.uint32(0), vu)
                cbuf[pl.ds(j * 16, 16)] = plsc.bitcast(vu, jnp.int32)
                return jnp.maximum(bc, vu)

            bc = lax.fori_loop(0, nv, scan_body, jnp.zeros((16,), jnp.uint32))
            best = jnp.max(bc)
            col = (best & jnp.uint32(0x1FFF)).astype(jnp.int32)
            hit0 = jnp.logical_and(iota16 == kk, kk < 16)
            hit1 = jnp.logical_and(iota16 == kk - 16, kk >= 16)
            si0 = jnp.where(hit0, col, si0)
            si1 = jnp.where(hit1, col, si1)
            return best, si0, si1

        _, si0, si1 = lax.fori_loop(
            0, _K, sel_body, (jnp.uint32(0), rsplat, rsplat))

        sel0 = plsc.load_gather(buf_sim, [si0])
        sel1 = jnp.where(iota16 < _K - 16,
                         plsc.load_gather(buf_sim, [si1]), _NEG)

        mm = jnp.maximum(jnp.max(sel0), jnp.max(sel1))
        e0 = jnp.exp(sel0 - mm)
        e1 = jnp.exp(sel1 - mm)
        ssum = jnp.sum(e0) + jnp.sum(e1)
        w0 = e0 / ssum
        w1 = e1 / ssum
        ws_new = tuple([w0[l] for l in range(16)] +
                       [w1[l] for l in range(_K - 16)])

        pltpu.async_copy(b_hbm.at[si0], bbufs[par].at[pl.ds(0, 16)],
                         semb[par][0])
        pltpu.async_copy(b_hbm.at[si1], bbufs[par].at[pl.ds(16, 16)],
                         semb[par][1])
        pltpu.async_copy(a_hbm.at[row], arows[par], sema[par])

        if par == 0:
            pl.when(i > 0)(lambda: agg_write(1, row - 1, ws))
        else:
            agg_write(0, row - 1, ws)
        ws = ws_new
      return ws

    ws_fin = lax.fori_loop(0, _RPW // 2, pair_body,
                           tuple(jnp.float32(0.0) for _ in range(_K)))
    agg_write(1, base + _RPW - 1, ws_fin)


def _sc_stage(sim, thr, cm, a, b):
    mesh = plsc.VectorSubcoreMesh(core_axis_name="c", subcore_axis_name="s")
    fn = functools.partial(
        pl.kernel,
        mesh=mesh,
        compiler_params=pltpu.CompilerParams(needs_layout_passes=False),
        out_type=jax.ShapeDtypeStruct((_N, _C), jnp.float32),
        scratch_types=[
            pltpu.VMEM((_N,), jnp.float32),
            pltpu.VMEM((_N,), jnp.float32),
            pltpu.VMEM((_RPW + 16,), jnp.float32),
            pltpu.VMEM((_NCHUNK,), jnp.float32),
            pltpu.VMEM((_NCHUNK,), jnp.float32),
            pltpu.VMEM((_C,), jnp.float32),
            pltpu.VMEM((_C,), jnp.float32),
            pltpu.VMEM((_N + 64,), jnp.int32),
            pltpu.VMEM((_GC, _C), jnp.float32),
            pltpu.VMEM((_GC, _C), jnp.float32),
            pltpu.VMEM((_C,), jnp.float32),
        ] + [pltpu.SemaphoreType.DMA] * 10,
    )(_sc_body)
    return fn(sim, thr, cm, a, b)


_UPD_BLK = 512


def _upd_body(x_ref, r_ref, u_ref, w2_ref, b2_ref, uw1b_ref, uw2_ref,
              ub2_ref, o_ref):
    h = jnp.dot(r_ref[...], w2_ref[...],
                preferred_element_type=jnp.float32) + b2_ref[...]
    t = jnp.maximum(
        u_ref[...] + jnp.dot(h, uw1b_ref[...],
                             preferred_element_type=jnp.float32), 0.0)
    o_ref[...] = x_ref[...] + jnp.dot(
        t, uw2_ref[...], preferred_element_type=jnp.float32) + ub2_ref[...]


def _upd(x, r, u, w2, b2, uw1b, uw2, ub2):
    row_spec = pl.BlockSpec((_UPD_BLK, _C), lambda i: (i, 0))
    w_spec = pl.BlockSpec((_C, _C), lambda i: (0, 0))
    b_spec = pl.BlockSpec((1, _C), lambda i: (0, 0))
    return pl.pallas_call(
        _upd_body,
        grid=(_N // _UPD_BLK,),
        in_specs=[row_spec, row_spec, row_spec, w_spec, b_spec, w_spec,
                  w_spec, b_spec],
        out_specs=row_spec,
        out_shape=jax.ShapeDtypeStruct((_N, _C), jnp.float32),
    )(x, r, u, w2, b2, uw1b, uw2, ub2)


def kernel(x, Wq, bq, Wk, bk, msg_W1, msg_b1, msg_W2, msg_b2,
           upd_W1, upd_b1, upd_W2, upd_b2):
    wcat = jnp.concatenate(
        [Wq, Wk, msg_W1[:_C], msg_W1[_C:], upd_W1[:_C]], axis=1)
    bcat = jnp.concatenate(
        [bq, bk, msg_b1, jnp.zeros_like(msg_b1), upd_b1]).reshape(1, 5 * _C)
    q, k, a, b, u = _prep(x, wcat, bcat)
    sim, thr, cm = _sim(k, q)
    r = _sc_stage(sim, thr.reshape(_N), cm, a, b)
    return _upd(x, r, u, msg_W2, msg_b2.reshape(1, _C), upd_W1[_C:],
                upd_W2, upd_b2.reshape(1, _C))


# speedup vs baseline: 2.7976x; 1.8220x over previous
import functools

import jax
import jax.numpy as jnp
from jax import lax
from jax.experimental import pallas as pl
from jax.experimental.pallas import tpu as pltpu
from jax.experimental.pallas import tpu_sc as plsc

_N = 8192
_C = 512
_K = 20
_NEG = -3.0e38


_PREP_BLK = 512


def _prep_body(x_ref, w_ref, b_ref, q_ref, k_ref, a_ref, bb_ref, u_ref):
    p = jnp.dot(x_ref[...], w_ref[...], preferred_element_type=jnp.float32)
    p = p + b_ref[...]
    q_ref[...] = p[:, 0 * _C:1 * _C]
    k_ref[...] = p[:, 1 * _C:2 * _C]
    a_ref[...] = p[:, 2 * _C:3 * _C]
    bb_ref[...] = p[:, 3 * _C:4 * _C]
    u_ref[...] = p[:, 4 * _C:5 * _C]


def _prep(x, wcat, bcat):
    n_out = 5
    outs = pl.pallas_call(
        _prep_body,
        grid=(_N // _PREP_BLK,),
        in_specs=[
            pl.BlockSpec((_PREP_BLK, _C), lambda i: (i, 0)),
            pl.BlockSpec((_C, n_out * _C), lambda i: (0, 0)),
            pl.BlockSpec((1, n_out * _C), lambda i: (0, 0)),
        ],
        out_specs=[pl.BlockSpec((_PREP_BLK, _C), lambda i: (i, 0))] * n_out,
        out_shape=[jax.ShapeDtypeStruct((_N, _C), jnp.float32)] * n_out,
    )(x, wcat, bcat)
    return outs


_SIM_RB = 256
_CHUNK = 128
_NCHUNK = _N // _CHUNK


def _sim_body(k_ref, q_ref, sim_ref, thr_ref, cmax_ref):
    s = lax.dot_general(
        k_ref[...], q_ref[...], (((1,), (1,)), ((), ())),
        preferred_element_type=jnp.float32)
    sim_ref[...] = s
    cm = jnp.concatenate(
        [jnp.max(s[:, c * _CHUNK:(c + 1) * _CHUNK], axis=1, keepdims=True)
         for c in range(_NCHUNK)], axis=1)
    cmax_ref[...] = cm
    for _ in range(_K - 1):
        m = jnp.max(cm, axis=1, keepdims=True)
        cm = jnp.where(cm >= m, _NEG, cm)
    thr_ref[...] = jnp.max(cm, axis=1, keepdims=True)


def _sim(k, q):
    return pl.pallas_call(
        _sim_body,
        grid=(_N // _SIM_RB,),
        in_specs=[
            pl.BlockSpec((_SIM_RB, _C), lambda i: (i, 0)),
            pl.BlockSpec((_N, _C), lambda i: (0, 0)),
        ],
        out_specs=[
            pl.BlockSpec((_SIM_RB, _N), lambda i: (i, 0)),
            pl.BlockSpec((_SIM_RB, 1), lambda i: (i, 0)),
            pl.BlockSpec((_SIM_RB, _NCHUNK), lambda i: (i, 0)),
        ],
        out_shape=[
            jax.ShapeDtypeStruct((_N, _N), jnp.float32),
            jax.ShapeDtypeStruct((_N, 1), jnp.float32),
            jax.ShapeDtypeStruct((_N, _NCHUNK), jnp.float32),
        ],
    )(k, q)


_NC = 2
_NS = 16
_NW = _NC * _NS
_RPW = _N // _NW
_GC = 32


def _sc_body(sim_hbm, thr_hbm, cm_hbm, a_hbm, b_hbm, r_hbm,
             rowb0, rowb1, thrbuf, cmb0, cmb1, arow0, arow1, cbuf, hitbuf,
             bbuf0, bbuf1, accbuf, semb00, semb01, semb10, semb11,
             semr0, semr1, semc0, semc1, sema0, sema1):
    wid = lax.axis_index("s") * _NC + lax.axis_index("c")
    base = wid * _RPW
    pltpu.sync_copy(thr_hbm.at[pl.ds(base, _RPW)], thrbuf.at[pl.ds(0, _RPW)])
    iota16 = lax.iota(jnp.int32, 16)
    rowbufs = (rowb0, rowb1)
    cmbufs = (cmb0, cmb1)
    arows = (arow0, arow1)
    bbufs = (bbuf0, bbuf1)
    semr = (semr0, semr1)
    semc = (semc0, semc1)
    sema = (sema0, sema1)
    semb = ((semb00, semb01), (semb10, semb11))
    dummy_idx = jnp.zeros((16,), jnp.int32)

    def agg_write(gpar, rowm1, ws):
        pltpu.make_async_copy(b_hbm.at[dummy_idx],
                              bbufs[gpar].at[pl.ds(0, 16)],
                              semb[gpar][0]).wait()
        pltpu.make_async_copy(b_hbm.at[dummy_idx],
                              bbufs[gpar].at[pl.ds(16, 16)],
                              semb[gpar][1]).wait()
        pltpu.make_async_copy(a_hbm.at[rowm1], arows[gpar],
                              sema[gpar]).wait()

        def ch_body(c2, _c):
            a = arows[gpar][pl.ds(c2 * 16, 16)]
            acc = jnp.zeros((16,), jnp.float32)
            for r2 in range(_K):
                b = bbufs[gpar][r2, pl.ds(c2 * 16, 16)]
                acc = acc + ws[r2] * jnp.maximum(a + b, 0.0)
            accbuf[pl.ds(c2 * 16, 16)] = acc
            return _c

        lax.fori_loop(0, _C // 16, ch_body, 0, unroll=2)
        pltpu.sync_copy(accbuf, r_hbm.at[rowm1])

    pltpu.async_copy(sim_hbm.at[base], rowbufs[0], semr[0])
    pltpu.async_copy(cm_hbm.at[base], cmbufs[0], semc[0])

    def pair_body(i, carry):
      ws = carry
      for par in (0, 1):
        row = base + 2 * i + par
        buf_sim = rowbufs[par]
        buf_cm = cmbufs[par]
        pltpu.make_async_copy(sim_hbm.at[row], buf_sim, semr[par]).wait()
        pltpu.make_async_copy(cm_hbm.at[row], buf_cm, semc[par]).wait()

        def _pref():
            pltpu.async_copy(sim_hbm.at[row + 1], rowbufs[1 - par],
                             semr[1 - par])
            pltpu.async_copy(cm_hbm.at[row + 1], cmbufs[1 - par],
                             semc[1 - par])
        if par == 0:
            _pref()
        else:
            pl.when(i < _RPW // 2 - 1)(_pref)

        thr_s = thrbuf[pl.ds(2 * i + par, 16)][0]

        dump = jnp.int32(_N + 40)
        hdump = jnp.int32(_NCHUNK + 40)

        hcnt_vec = jnp.zeros((16,), jnp.int32)
        for g in range(_NCHUNK // 16):
            cmv = buf_cm[pl.ds(g * 16, 16)]
            hmsk = cmv >= thr_s
            hones = jnp.where(hmsk, jnp.int32(1), jnp.int32(0))
            hcs = plsc.cumsum(hones)
            hpos = jnp.where(hmsk, hcnt_vec + hcs - 1, hdump)
            plsc.store_scatter(hitbuf, [hpos], g * 16 + iota16)
            hcnt_vec = hcnt_vec + plsc.all_reduce_population_count(hmsk)
        hcnt = hcnt_vec[0]

        def cbody(h, cnt_vec):
            cc = hitbuf[pl.ds(h, 16)][0]
            for j in range(_CHUNK // 16):
                v = buf_sim[pl.ds(cc * _CHUNK + j * 16, 16)]
                msk = v >= thr_s
                mono = plsc.bitcast(v, jnp.uint32) ^ jnp.where(
                    v < 0.0, jnp.uint32(0xFFFFFFFF), jnp.uint32(0x80000000))
                col = cc * _CHUNK + j * 16 + iota16
                comb = (mono & jnp.uint32(0xFFFFE000)) | col.astype(
                    jnp.uint32)
                ones = jnp.where(msk, jnp.int32(1), jnp.int32(0))
                cs = plsc.cumsum(ones)
                pos = jnp.where(msk, cnt_vec + cs - 1, dump)
                plsc.store_scatter(cbuf, [pos], plsc.bitcast(comb, jnp.int32))
                cnt_vec = cnt_vec + plsc.all_reduce_population_count(msk)
            return cnt_vec

        cnt_vec = lax.fori_loop(0, hcnt, cbody, jnp.zeros((16,), jnp.int32))
        cnt = cnt_vec[0]
        cbuf[pl.ds(cnt, 16)] = jnp.zeros((16,), jnp.int32)
        nv = cnt // 16 + 1

        rsplat = jnp.zeros((16,), jnp.int32) + row

        def sel_body(kk, carry):
            last, si0, si1 = carry

            def scan_body(j, bc):
                vu = plsc.bitcast(cbuf[pl.ds(j * 16, 16)], jnp.uint32)
                vu = jnp.where(vu == last, jnp.uint32(0), vu)
                cbuf[pl.ds(j * 16, 16)] = plsc.bitcast(vu, jnp.int32)
                return jnp.maximum(bc, vu)

            bc = lax.fori_loop(0, nv, scan_body, jnp.zeros((16,), jnp.uint32))
            best = jnp.max(bc)
            col = (best & jnp.uint32(0x1FFF)).astype(jnp.int32)
            hit0 = jnp.logical_and(iota16 == kk, kk < 16)
            hit1 = jnp.logical_and(iota16 == kk - 16, kk >= 16)
            si0 = jnp.where(hit0, col, si0)
            si1 = jnp.where(hit1, col, si1)
            return best, si0, si1

        _, si0, si1 = lax.fori_loop(
            0, _K, sel_body, (jnp.uint32(0), rsplat, rsplat))

        sel0 = plsc.load_gather(buf_sim, [si0])
        sel1 = jnp.where(iota16 < _K - 16,
                         plsc.load_gather(buf_sim, [si1]), _NEG)

        mm = jnp.maximum(jnp.max(sel0), jnp.max(sel1))
        e0 = jnp.exp(sel0 - mm)
        e1 = jnp.exp(sel1 - mm)
        ssum = jnp.sum(e0) + jnp.sum(e1)
        w0 = e0 / ssum
        w1 = e1 / ssum
        ws_new = tuple([w0[l] for l in range(16)] +
                       [w1[l] for l in range(_K - 16)])

        pltpu.async_copy(b_hbm.at[si0], bbufs[par].at[pl.ds(0, 16)],
                         semb[par][0])
        pltpu.async_copy(b_hbm.at[si1], bbufs[par].at[pl.ds(16, 16)],
                         semb[par][1])
        pltpu.async_copy(a_hbm.at[row], arows[par], sema[par])

        if par == 0:
            pl.when(i > 0)(lambda: agg_write(1, row - 1, ws))
        else:
            agg_write(0, row - 1, ws)
        ws = ws_new
      return ws

    ws_fin = lax.fori_loop(0, _RPW // 2, pair_body,
                           tuple(jnp.float32(0.0) for _ in range(_K)))
    agg_write(1, base + _RPW - 1, ws_fin)


def _sc_stage(sim, thr, cm, a, b):
    mesh = plsc.VectorSubcoreMesh(core_axis_name="c", subcore_axis_name="s")
    fn = functools.partial(
        pl.kernel,
        mesh=mesh,
        compiler_params=pltpu.CompilerParams(needs_layout_passes=False),
        out_type=jax.ShapeDtypeStruct((_N, _C), jnp.float32),
        scratch_types=[
            pltpu.VMEM((_N,), jnp.float32),
            pltpu.VMEM((_N,), jnp.float32),
            pltpu.VMEM((_RPW + 16,), jnp.float32),
            pltpu.VMEM((_NCHUNK,), jnp.float32),
            pltpu.VMEM((_NCHUNK,), jnp.float32),
            pltpu.VMEM((_C,), jnp.float32),
            pltpu.VMEM((_C,), jnp.float32),
            pltpu.VMEM((_N + 64,), jnp.int32),
            pltpu.VMEM((_NCHUNK + 64,), jnp.int32),
            pltpu.VMEM((_GC, _C), jnp.float32),
            pltpu.VMEM((_GC, _C), jnp.float32),
            pltpu.VMEM((_C,), jnp.float32),
        ] + [pltpu.SemaphoreType.DMA] * 10,
    )(_sc_body)
    return fn(sim, thr, cm, a, b)


_UPD_BLK = 512


def _upd_body(x_ref, r_ref, u_ref, w2_ref, b2_ref, uw1b_ref, uw2_ref,
              ub2_ref, o_ref):
    h = jnp.dot(r_ref[...], w2_ref[...],
                preferred_element_type=jnp.float32) + b2_ref[...]
    t = jnp.maximum(
        u_ref[...] + jnp.dot(h, uw1b_ref[...],
                             preferred_element_type=jnp.float32), 0.0)
    o_ref[...] = x_ref[...] + jnp.dot(
        t, uw2_ref[...], preferred_element_type=jnp.float32) + ub2_ref[...]


def _upd(x, r, u, w2, b2, uw1b, uw2, ub2):
    row_spec = pl.BlockSpec((_UPD_BLK, _C), lambda i: (i, 0))
    w_spec = pl.BlockSpec((_C, _C), lambda i: (0, 0))
    b_spec = pl.BlockSpec((1, _C), lambda i: (0, 0))
    return pl.pallas_call(
        _upd_body,
        grid=(_N // _UPD_BLK,),
        in_specs=[row_spec, row_spec, row_spec, w_spec, b_spec, w_spec,
                  w_spec, b_spec],
        out_specs=row_spec,
        out_shape=jax.ShapeDtypeStruct((_N, _C), jnp.float32),
    )(x, r, u, w2, b2, uw1b, uw2, ub2)


def kernel(x, Wq, bq, Wk, bk, msg_W1, msg_b1, msg_W2, msg_b2,
           upd_W1, upd_b1, upd_W2, upd_b2):
    wcat = jnp.concatenate(
        [Wq, Wk, msg_W1[:_C], msg_W1[_C:], upd_W1[:_C]], axis=1)
    bcat = jnp.concatenate(
        [bq, bk, msg_b1, jnp.zeros_like(msg_b1), upd_b1]).reshape(1, 5 * _C)
    q, k, a, b, u = _prep(x, wcat, bcat)
    sim, thr, cm = _sim(k, q)
    r = _sc_stage(sim, thr.reshape(_N), cm, a, b)
    return _upd(x, r, u, msg_W2, msg_b2.reshape(1, _C), upd_W1[_C:],
                upd_W2, upd_b2.reshape(1, _C))
